# baseline (speedup 1.0000x reference)
import functools

import jax
import jax.numpy as jnp
from jax import lax
from jax.experimental import pallas as pl
from jax.experimental.pallas import tpu as pltpu
from jax.experimental.pallas import tpu_sc as plsc

_NC = 2
_NS = 16
_NW = _NC * _NS
_CH = 128


def _project_kernel(f_ref, w_ref, b_ref, y_ref):
    y = jnp.dot(f_ref[...], w_ref[...],
                preferred_element_type=jnp.float32,
                precision=lax.Precision.HIGHEST)
    y = y + b_ref[...]
    nk, _, d = y_ref.shape
    for k in range(nk):
        y_ref[k] = y[:, k * d:(k + 1) * d]


def _build_table(feat, w_cat, b_cat, nk, bn):
    n, in_feat = feat.shape
    kout = w_cat.shape[1]
    d = kout // nk
    return pl.pallas_call(
        _project_kernel,
        grid=(n // bn,),
        in_specs=[
            pl.BlockSpec((bn, in_feat), lambda i: (i, 0)),
            pl.BlockSpec((in_feat, kout), lambda i: (0, 0)),
            pl.BlockSpec((1, kout), lambda i: (0, 0)),
        ],
        out_specs=pl.BlockSpec((nk, bn, d), lambda i: (0, i, 0)),
        out_shape=jax.ShapeDtypeStruct((nk, n, d), jnp.float32),
    )(feat, w_cat, b_cat)


def _combine_kernel(p_ref, o_ref):
    o_ref[...] = p_ref[0] + p_ref[1]


def _combine(partials, bn):
    _, n, out_feat = partials.shape
    return pl.pallas_call(
        _combine_kernel,
        grid=(n // bn,),
        in_specs=[pl.BlockSpec((2, bn, out_feat), lambda i: (0, i, 0))],
        out_specs=pl.BlockSpec((bn, out_feat), lambda i: (i, 0)),
        out_shape=jax.ShapeDtypeStruct((n, out_feat), jnp.float32),
    )(partials)


def _make_edge_kernel(out_feat, nch, acc_rows, rows_per_tile):
    mesh = plsc.VectorSubcoreMesh(core_axis_name="c", subcore_axis_name="s")

    @functools.partial(
        pl.kernel,
        mesh=mesh,
        out_type=jax.ShapeDtypeStruct((_NC, acc_rows, out_feat), jnp.float32),
        scratch_types=[
            pltpu.VMEM((nch // 2, _CH), jnp.int32),
            pltpu.VMEM((nch // 2, _CH), jnp.int32),
            pltpu.VMEM((_CH, out_feat), jnp.float32),
            pltpu.VMEM((_CH, out_feat), jnp.float32),
            pltpu.VMEM_SHARED((acc_rows, out_feat), jnp.float32),
            pltpu.SemaphoreType.DMA,
            pltpu.SemaphoreType.DMA,
        ],
    )
    def edge_kernel(table_hbm, gidx_hbm, didx_hbm, zeros_hbm, out_hbm,
                    gidx_v, didx_v, rows0, rows1, acc, sem0, sem1):
        c = lax.axis_index("c")
        s = lax.axis_index("s")
        wid = s * _NC + c

        z0 = s * rows_per_tile
        pltpu.sync_copy(zeros_hbm.at[pl.ds(0, rows_per_tile)],
                        acc.at[pl.ds(z0, rows_per_tile)])
        plsc.subcore_barrier()

        nh = nch // 2

        def _gather(j, buf):
            pltpu.sync_copy(table_hbm.at[gidx_v.at[j]], buf)

        def _scat_start(j, buf, sem):
            pltpu.async_copy(buf, acc.at[didx_v.at[j]], sem, add=True)

        def _scat_wait(j, buf, sem):
            pltpu.make_async_copy(buf, acc.at[didx_v.at[j]], sem).wait()

        with jax.named_scope("edge_loop"):
            for h in range(2):
                with jax.named_scope("stage_idx"):
                    pltpu.sync_copy(gidx_hbm.at[wid, h], gidx_v)
                    pltpu.sync_copy(didx_hbm.at[wid, h], didx_v)
                _gather(0, rows0)
                _scat_start(0, rows0, sem0)
                _gather(1, rows1)
                _scat_start(1, rows1, sem1)

                def body(t, carry):
                    j = 2 * t + 2
                    _scat_wait(j - 2, rows0, sem0)
                    _gather(j, rows0)
                    _scat_start(j, rows0, sem0)
                    _scat_wait(j - 1, rows1, sem1)
                    _gather(j + 1, rows1)
                    _scat_start(j + 1, rows1, sem1)
                    return carry

                lax.fori_loop(0, (nh - 2) // 2, body, 0)
                _scat_wait(0, rows0, sem0)
                _scat_wait(1, rows1, sem1)
        with jax.named_scope("post_barrier"):
            plsc.subcore_barrier()

        r0 = s * rows_per_tile
        pltpu.sync_copy(acc.at[pl.ds(r0, rows_per_tile)],
                        out_hbm.at[c, pl.ds(r0, rows_per_tile)])

    return edge_kernel


def kernel(feat, edge_index, etypes, ntypes, weight, m_bias, loop_weight, h_bias):
    n, in_feat = feat.shape
    num_rels, _, out_feat = weight.shape
    num_nt = loop_weight.shape[0]
    e = edge_index.shape[1]

    nk = num_rels + num_nt
    w_cat = jnp.concatenate([weight, loop_weight], axis=0)
    w_cat = w_cat.transpose(1, 0, 2).reshape(in_feat, nk * out_feat)
    b_cat = jnp.concatenate([m_bias, h_bias[:, 0, :]], axis=0).reshape(1, -1)

    src = edge_index[0].astype(jnp.int32)
    dst = edge_index[1].astype(jnp.int32)
    node_ids = jnp.arange(n, dtype=jnp.int32)

    rows_per_tile = -(-(-(-n // _NS)) // 8) * 8
    acc_rows = _NS * rows_per_tile
    zeros = jnp.zeros((rows_per_tile, out_feat), jnp.float32)

    def _shard(g, d, per_w):
        total = _NW * per_w
        p = total - g.shape[0]
        g = jnp.concatenate(
            [g, jnp.zeros((p,), jnp.int32)]) if p else g
        d = jnp.concatenate(
            [d, n + jnp.arange(p, dtype=jnp.int32) % (acc_rows - n)]) if p else d
        return g.reshape(_NW, per_w), d.reshape(_NW, per_w)

    rpw = -(-e // _NW)
    vpw = -(-n // _NW)
    pw = -(-(rpw + vpw) // (4 * _CH)) * 4 * _CH
    nch = pw // _CH
    rg, rd = _shard(etypes.astype(jnp.int32) * n + src, dst, rpw)
    vg, vd = _shard((num_rels + ntypes.astype(jnp.int32)) * n + node_ids,
                    node_ids, vpw)
    tail = pw - rpw - vpw
    tg = jnp.zeros((_NW, tail), jnp.int32)
    td = jnp.broadcast_to(
        n + jnp.arange(tail, dtype=jnp.int32) % (acc_rows - n), (_NW, tail))
    gidx = jnp.concatenate([rg, vg, tg], axis=1).reshape(_NW, 2, nch // 2, _CH)
    didx = jnp.concatenate([rd, vd, td], axis=1).reshape(_NW, 2, nch // 2, _CH)

    table = _build_table(feat, w_cat, b_cat, nk, bn=2000).reshape(-1, out_feat)

    edge_kernel = _make_edge_kernel(out_feat, nch, acc_rows, rows_per_tile)
    partials = edge_kernel(table, gidx, didx, zeros)[:, :n, :]

    out = _combine(partials, bn=1000)
    return out[:, None, :]


# speedup vs baseline: 2.1977x; 2.1977x over previous
import functools

import jax
import jax.numpy as jnp
from jax import lax
from jax.experimental import pallas as pl
from jax.experimental.pallas import tpu as pltpu
from jax.experimental.pallas import tpu_sc as plsc

_NC = 2
_NS = 16
_NW = _NC * _NS
_CH = 128


def _project_kernel(f_ref, w_ref, b_ref, y_ref):
    y = jnp.dot(f_ref[...], w_ref[...],
                preferred_element_type=jnp.float32,
                precision=lax.Precision.HIGHEST)
    y_ref[...] = y + b_ref[...]


def _build_table(feat, w_cat, b_cat, bn):
    n, in_feat = feat.shape
    kout = w_cat.shape[1]
    return pl.pallas_call(
        _project_kernel,
        grid=(n // bn,),
        in_specs=[
            pl.BlockSpec((bn, in_feat), lambda i: (i, 0)),
            pl.BlockSpec((in_feat, kout), lambda i: (0, 0)),
            pl.BlockSpec((1, kout), lambda i: (0, 0)),
        ],
        out_specs=pl.BlockSpec((bn, kout), lambda i: (i, 0)),
        out_shape=jax.ShapeDtypeStruct((n, kout), jnp.float32),
    )(feat, w_cat, b_cat)


def _combine_kernel(p_ref, o_ref):
    o_ref[...] = p_ref[0] + p_ref[1]


def _combine(partials, bn):
    _, n, out_feat = partials.shape
    return pl.pallas_call(
        _combine_kernel,
        grid=(n // bn,),
        in_specs=[pl.BlockSpec((2, bn, out_feat), lambda i: (0, i, 0))],
        out_specs=pl.BlockSpec((bn, out_feat), lambda i: (i, 0)),
        out_shape=jax.ShapeDtypeStruct((n, out_feat), jnp.float32),
    )(partials)


def _make_edge_kernel(out_feat, nch, acc_rows, rows_per_tile):
    mesh = plsc.VectorSubcoreMesh(core_axis_name="c", subcore_axis_name="s")

    @functools.partial(
        pl.kernel,
        mesh=mesh,
        out_type=jax.ShapeDtypeStruct((_NC, acc_rows, out_feat), jnp.float32),
        scratch_types=[
            pltpu.VMEM((nch, _CH), jnp.int32),
            pltpu.VMEM((nch, _CH), jnp.int32),
            pltpu.VMEM((_CH, out_feat), jnp.float32),
            pltpu.VMEM_SHARED((acc_rows, out_feat), jnp.float32),
            pltpu.SemaphoreType.DMA,
        ],
    )
    def edge_kernel(table_hbm, gidx_hbm, didx_hbm, zeros_hbm, out_hbm,
                    gidx_v, didx_v, rows0, acc, sem0):
        c = lax.axis_index("c")
        s = lax.axis_index("s")
        wid = s * _NC + c

        z0 = s * rows_per_tile
        pltpu.sync_copy(zeros_hbm.at[pl.ds(0, rows_per_tile)],
                        acc.at[pl.ds(z0, rows_per_tile)])
        pltpu.sync_copy(gidx_hbm.at[wid], gidx_v)
        pltpu.sync_copy(didx_hbm.at[wid], didx_v)
        plsc.subcore_barrier()

        def body(j, carry):
            pltpu.async_copy(table_hbm.at[gidx_v.at[j]], rows0, sem0).wait()
            pltpu.sync_copy(rows0, acc.at[didx_v.at[j]], add=True)
            return carry

        lax.fori_loop(0, nch, body, 0)
        plsc.subcore_barrier()

        r0 = s * rows_per_tile
        pltpu.sync_copy(acc.at[pl.ds(r0, rows_per_tile)],
                        out_hbm.at[c, pl.ds(r0, rows_per_tile)])

    return edge_kernel


def kernel(feat, edge_index, etypes, ntypes, weight, m_bias, loop_weight, h_bias):
    n, in_feat = feat.shape
    num_rels, _, out_feat = weight.shape
    num_nt = loop_weight.shape[0]
    e = edge_index.shape[1]

    nk = num_rels + num_nt
    w_cat = jnp.concatenate([weight, loop_weight], axis=0)
    w_cat = w_cat.transpose(1, 0, 2).reshape(in_feat, nk * out_feat)
    b_cat = jnp.concatenate([m_bias, h_bias[:, 0, :]], axis=0).reshape(1, -1)

    src = edge_index[0].astype(jnp.int32)
    dst = edge_index[1].astype(jnp.int32)
    node_ids = jnp.arange(n, dtype=jnp.int32)

    rows_per_tile = -(-(-(-n // _NS)) // 8) * 8
    acc_rows = _NS * rows_per_tile
    zeros = jnp.zeros((rows_per_tile, out_feat), jnp.float32)

    def _padg(p):
        return (jnp.arange(p, dtype=jnp.int32) * 97) % (nk * n)

    def _shard(g, d, per_w):
        total = _NW * per_w
        p = total - g.shape[0]
        g = jnp.concatenate([g, _padg(p)]) if p else g
        d = jnp.concatenate(
            [d, n + jnp.arange(p, dtype=jnp.int32) % (acc_rows - n)]) if p else d
        return g.reshape(_NW, per_w), d.reshape(_NW, per_w)

    rpw = -(-e // _NW)
    vpw = -(-n // _NW)
    pw = -(-(rpw + vpw) // (2 * _CH)) * 2 * _CH
    nch = pw // _CH
    rg, rd = _shard(src * nk + etypes.astype(jnp.int32), dst, rpw)
    vg, vd = _shard(node_ids * nk + (num_rels + ntypes.astype(jnp.int32)),
                    node_ids, vpw)
    tail = pw - rpw - vpw
    tg = _padg(_NW * tail).reshape(_NW, tail)
    td = jnp.broadcast_to(
        n + jnp.arange(tail, dtype=jnp.int32) % (acc_rows - n), (_NW, tail))
    gidx = jnp.concatenate([rg, vg, tg], axis=1).reshape(_NW, nch, _CH)
    didx = jnp.concatenate([rd, vd, td], axis=1).reshape(_NW, nch, _CH)

    table = _build_table(feat, w_cat, b_cat, bn=2000).reshape(-1, out_feat)

    edge_kernel = _make_edge_kernel(out_feat, nch, acc_rows, rows_per_tile)
    partials = edge_kernel(table, gidx, didx, zeros)[:, :n, :]

    out = _combine(partials, bn=1000)
    return out[:, None, :]


# speedup vs baseline: 2.6733x; 1.2164x over previous
import functools

import jax
import jax.numpy as jnp
from jax import lax
from jax.experimental import pallas as pl
from jax.experimental.pallas import tpu as pltpu
from jax.experimental.pallas import tpu_sc as plsc

_NC = 2
_NS = 16
_NW = _NC * _NS
_CH = 128


def _project_kernel(f_ref, w_ref, b_ref, y_ref):
    y = jnp.dot(f_ref[...], w_ref[...],
                preferred_element_type=jnp.float32,
                precision=lax.Precision.HIGHEST)
    y = y + b_ref[...]
    nk, _, d = y_ref.shape
    for k in range(nk):
        y_ref[k] = y[:, k * d:(k + 1) * d]


def _build_table(feat, w_cat, b_cat, nk, bn):
    n, in_feat = feat.shape
    kout = w_cat.shape[1]
    d = kout // nk
    return pl.pallas_call(
        _project_kernel,
        grid=(n // bn,),
        in_specs=[
            pl.BlockSpec((bn, in_feat), lambda i: (i, 0)),
            pl.BlockSpec((in_feat, kout), lambda i: (0, 0)),
            pl.BlockSpec((1, kout), lambda i: (0, 0)),
        ],
        out_specs=pl.BlockSpec((nk, bn, d), lambda i: (0, i, 0)),
        out_shape=jax.ShapeDtypeStruct((nk, n, d), jnp.float32),
    )(feat, w_cat, b_cat)


def _combine_kernel(p_ref, o_ref):
    o_ref[...] = p_ref[0] + p_ref[1]


def _combine(partials, bn):
    _, n, out_feat = partials.shape
    return pl.pallas_call(
        _combine_kernel,
        grid=(n // bn,),
        in_specs=[pl.BlockSpec((2, bn, out_feat), lambda i: (0, i, 0))],
        out_specs=pl.BlockSpec((bn, out_feat), lambda i: (i, 0)),
        out_shape=jax.ShapeDtypeStruct((n, out_feat), jnp.float32),
    )(partials)


def _make_edge_kernel(out_feat, nch, acc_rows, rows_per_tile):
    mesh = plsc.VectorSubcoreMesh(core_axis_name="c", subcore_axis_name="s")

    @functools.partial(
        pl.kernel,
        mesh=mesh,
        out_type=jax.ShapeDtypeStruct((_NC, acc_rows, out_feat), jnp.float32),
        scratch_types=[
            pltpu.VMEM((nch, _CH), jnp.int32),
            pltpu.VMEM((nch, _CH), jnp.int32),
            pltpu.VMEM((_CH, out_feat), jnp.float32),
            pltpu.VMEM_SHARED((acc_rows, out_feat), jnp.float32),
            pltpu.SemaphoreType.DMA,
        ],
    )
    def edge_kernel(table_hbm, gidx_hbm, didx_hbm, zeros_hbm, out_hbm,
                    gidx_v, didx_v, rows0, acc, sem0):
        c = lax.axis_index("c")
        s = lax.axis_index("s")
        wid = s * _NC + c

        z0 = s * rows_per_tile
        pltpu.sync_copy(zeros_hbm.at[pl.ds(0, rows_per_tile)],
                        acc.at[pl.ds(z0, rows_per_tile)])
        pltpu.sync_copy(gidx_hbm.at[wid], gidx_v)
        pltpu.sync_copy(didx_hbm.at[wid], didx_v)
        plsc.subcore_barrier()

        def body(j, carry):
            pltpu.async_copy(table_hbm.at[gidx_v.at[j]], rows0, sem0).wait()
            pltpu.sync_copy(rows0, acc.at[didx_v.at[j]], add=True)
            return carry

        lax.fori_loop(0, nch, body, 0)
        plsc.subcore_barrier()

        r0 = s * rows_per_tile
        pltpu.sync_copy(acc.at[pl.ds(r0, rows_per_tile)],
                        out_hbm.at[c, pl.ds(r0, rows_per_tile)])

    return edge_kernel


def kernel(feat, edge_index, etypes, ntypes, weight, m_bias, loop_weight, h_bias):
    n, in_feat = feat.shape
    num_rels, _, out_feat = weight.shape
    num_nt = loop_weight.shape[0]
    e = edge_index.shape[1]

    nk = num_rels + num_nt
    w_cat = jnp.concatenate([weight, loop_weight], axis=0)
    w_cat = w_cat.transpose(1, 0, 2).reshape(in_feat, nk * out_feat)
    b_cat = jnp.concatenate([m_bias, h_bias[:, 0, :]], axis=0).reshape(1, -1)

    src = edge_index[0].astype(jnp.int32)
    dst = edge_index[1].astype(jnp.int32)
    node_ids = jnp.arange(n, dtype=jnp.int32)

    rows_per_tile = -(-(-(-n // _NS)) // 8) * 8
    acc_rows = _NS * rows_per_tile
    zeros = jnp.zeros((rows_per_tile, out_feat), jnp.float32)

    def _padg(p):
        return (jnp.arange(p, dtype=jnp.int32) * 97) % (nk * n)

    def _shard(g, d, per_w):
        total = _NW * per_w
        p = total - g.shape[0]
        g = jnp.concatenate([g, _padg(p)]) if p else g
        d = jnp.concatenate(
            [d, n + jnp.arange(p, dtype=jnp.int32) % (acc_rows - n)]) if p else d
        return g.reshape(_NW, per_w), d.reshape(_NW, per_w)

    rpw = -(-e // _NW)
    vpw = -(-n // _NW)
    pw = -(-(rpw + vpw) // (2 * _CH)) * 2 * _CH
    nch = pw // _CH
    rg, rd = _shard(etypes.astype(jnp.int32) * n + src, dst, rpw)
    vg, vd = _shard((num_rels + ntypes.astype(jnp.int32)) * n + node_ids,
                    node_ids, vpw)
    tail = pw - rpw - vpw
    tg = _padg(_NW * tail).reshape(_NW, tail)
    td = jnp.broadcast_to(
        n + jnp.arange(tail, dtype=jnp.int32) % (acc_rows - n), (_NW, tail))
    gidx = jnp.concatenate([rg, vg, tg], axis=1).reshape(_NW, nch, _CH)
    didx = jnp.concatenate([rd, vd, td], axis=1).reshape(_NW, nch, _CH)

    table = _build_table(feat, w_cat, b_cat, nk, bn=2000).reshape(-1, out_feat)

    edge_kernel = _make_edge_kernel(out_feat, nch, acc_rows, rows_per_tile)
    partials = edge_kernel(table, gidx, didx, zeros)

    out = _combine(partials, bn=acc_rows // 8)
    return out[:n, None, :]


# speedup vs baseline: 3.0353x; 1.1354x over previous
import functools

import jax
import jax.numpy as jnp
from jax import lax
from jax.experimental import pallas as pl
from jax.experimental.pallas import tpu as pltpu
from jax.experimental.pallas import tpu_sc as plsc

_NC = 2
_NS = 16
_NW = _NC * _NS
_CH = 128


def _project_kernel(f_ref, w_ref, b_ref, y_ref):
    y = jnp.dot(f_ref[...], w_ref[...],
                preferred_element_type=jnp.float32,
                precision=lax.Precision.DEFAULT)
    y = y + b_ref[...]
    nk, _, d = y_ref.shape
    for k in range(nk):
        y_ref[k] = y[:, k * d:(k + 1) * d]


def _build_table(feat, w_cat, b_cat, nk, bn):
    n, in_feat = feat.shape
    kout = w_cat.shape[1]
    d = kout // nk
    return pl.pallas_call(
        _project_kernel,
        grid=(n // bn,),
        in_specs=[
            pl.BlockSpec((bn, in_feat), lambda i: (i, 0)),
            pl.BlockSpec((in_feat, kout), lambda i: (0, 0)),
            pl.BlockSpec((1, kout), lambda i: (0, 0)),
        ],
        out_specs=pl.BlockSpec((nk, bn, d), lambda i: (0, i, 0)),
        out_shape=jax.ShapeDtypeStruct((nk, n, d), jnp.float32),
    )(feat, w_cat, b_cat)


def _combine_kernel(p_ref, o_ref):
    o_ref[...] = p_ref[0] + p_ref[1]


def _combine(partials, n, bn):
    out_feat = partials.shape[2]
    return pl.pallas_call(
        _combine_kernel,
        grid=(n // bn,),
        in_specs=[pl.BlockSpec((2, bn, out_feat), lambda i: (0, i, 0))],
        out_specs=pl.BlockSpec((bn, out_feat), lambda i: (i, 0)),
        out_shape=jax.ShapeDtypeStruct((n, out_feat), jnp.float32),
    )(partials)


def _make_edge_kernel(out_feat, nch, acc_rows, rows_per_tile):
    mesh = plsc.VectorSubcoreMesh(core_axis_name="c", subcore_axis_name="s")

    @functools.partial(
        pl.kernel,
        mesh=mesh,
        out_type=jax.ShapeDtypeStruct((_NC, acc_rows, out_feat), jnp.float32),
        scratch_types=[
            pltpu.VMEM((nch, _CH), jnp.int32),
            pltpu.VMEM((nch, _CH), jnp.int32),
            pltpu.VMEM((_CH, out_feat), jnp.float32),
            pltpu.VMEM_SHARED((acc_rows, out_feat), jnp.float32),
            pltpu.SemaphoreType.DMA,
        ],
    )
    def edge_kernel(table_hbm, gidx_hbm, didx_hbm, zeros_hbm, out_hbm,
                    gidx_v, didx_v, rows0, acc, sem0):
        c = lax.axis_index("c")
        s = lax.axis_index("s")
        wid = s * _NC + c

        z0 = s * rows_per_tile
        pltpu.sync_copy(zeros_hbm.at[pl.ds(0, rows_per_tile)],
                        acc.at[pl.ds(z0, rows_per_tile)])
        pltpu.sync_copy(gidx_hbm.at[wid], gidx_v)
        pltpu.sync_copy(didx_hbm.at[wid], didx_v)
        plsc.subcore_barrier()

        def body(j, carry):
            pltpu.async_copy(table_hbm.at[gidx_v.at[j]], rows0, sem0).wait()
            pltpu.sync_copy(rows0, acc.at[didx_v.at[j]], add=True)
            return carry

        lax.fori_loop(0, nch, body, 0)
        plsc.subcore_barrier()

        r0 = s * rows_per_tile
        pltpu.sync_copy(acc.at[pl.ds(r0, rows_per_tile)],
                        out_hbm.at[c, pl.ds(r0, rows_per_tile)])

    return edge_kernel


def kernel(feat, edge_index, etypes, ntypes, weight, m_bias, loop_weight, h_bias):
    n, in_feat = feat.shape
    num_rels, _, out_feat = weight.shape
    num_nt = loop_weight.shape[0]
    e = edge_index.shape[1]

    nk = num_rels + num_nt
    w_cat = jnp.concatenate([weight, loop_weight], axis=0)
    w_cat = w_cat.transpose(1, 0, 2).reshape(in_feat, nk * out_feat)
    b_cat = jnp.concatenate([m_bias, h_bias[:, 0, :]], axis=0).reshape(1, -1)

    src = edge_index[0].astype(jnp.int32)
    dst = edge_index[1].astype(jnp.int32)
    node_ids = jnp.arange(n, dtype=jnp.int32)

    rows_per_tile = -(-(-(-n // _NS)) // 8) * 8
    acc_rows = _NS * rows_per_tile
    zeros = jnp.zeros((rows_per_tile, out_feat), jnp.float32)

    def _padg(p):
        return (jnp.arange(p, dtype=jnp.int32) * 97) % (nk * n)

    def _shard(g, d, per_w):
        total = _NW * per_w
        p = total - g.shape[0]
        g = jnp.concatenate([g, _padg(p)]) if p else g
        d = jnp.concatenate(
            [d, n + jnp.arange(p, dtype=jnp.int32) % (acc_rows - n)]) if p else d
        return g.reshape(_NW, per_w), d.reshape(_NW, per_w)

    rpw = -(-e // _NW)
    vpw = -(-n // _NW)
    pw = -(-(rpw + vpw) // (2 * _CH)) * 2 * _CH
    nch = pw // _CH
    rg, rd = _shard(etypes.astype(jnp.int32) * n + src, dst, rpw)
    vg, vd = _shard((num_rels + ntypes.astype(jnp.int32)) * n + node_ids,
                    node_ids, vpw)
    tail = pw - rpw - vpw
    tg = _padg(_NW * tail).reshape(_NW, tail)
    td = jnp.broadcast_to(
        n + jnp.arange(tail, dtype=jnp.int32) % (acc_rows - n), (_NW, tail))
    gidx = jnp.concatenate([rg, vg, tg], axis=1).reshape(_NW, nch, _CH)
    didx = jnp.concatenate([rd, vd, td], axis=1).reshape(_NW, nch, _CH)

    table = _build_table(feat, w_cat, b_cat, nk, bn=2000).reshape(-1, out_feat)

    edge_kernel = _make_edge_kernel(out_feat, nch, acc_rows, rows_per_tile)
    partials = edge_kernel(table, gidx, didx, zeros)

    out = _combine(partials, n, bn=1000)
    return out[:, None, :]
